# Initial kernel scaffold; baseline (speedup 1.0000x reference)
import jax
import jax.numpy as jnp
from jax.experimental import pallas as pl


def kernel(x, enc_w1, enc_b1, enc_w2, enc_b2, gcn1_w, gcn1_b, gcn2_w, gcn2_b, dec_w1, dec_b1, dec_w2, dec_b2):
    raise NotImplementedError("write your pallas kernel here")


# speedup vs baseline: 3.7786x; 3.7786x over previous
import jax
import jax.numpy as jnp
import numpy as np
from jax.experimental import pallas as pl
from jax.experimental.pallas import tpu as pltpu

_TB = 512


def _gae_kernel(xT_ref, encw1r_ref, encc_ref, encw2t_ref, encb2_ref,
                g1w_ref, g1b_ref,
                g2wt_ref, g2b_ref, dw1t_ref, db1_ref, dw2c_ref, db2c_ref,
                rec_ref, lat_ref, adj_ref):
    xT = xT_ref[...]
    TB = xT.shape[1]

    w1r = encw1r_ref[...]
    encc = encc_ref[...]
    H = jnp.maximum(xT[:, None, :] * w1r[0][None, :, None]
                    + encc[:, :, None], 0.0)
    encw2t = encw2t_ref[...]
    encb2 = encb2_ref[...]
    px = jnp.sum(H * encw2t[0][None, :, None], axis=1) + encb2[0, 0]
    py = jnp.sum(H * encw2t[1][None, :, None], axis=1) + encb2[1, 0]

    dx = px[:, None, :] - px[None, :, :]
    dy = py[:, None, :] - py[None, :, :]
    r2 = (dx * dx + dy * dy) * 0.25
    mx = (px[:, None, :] + px[None, :, :]) * 0.5
    my = (py[:, None, :] + py[None, :, :]) * 0.5
    ex = px[:, None, None, :] - mx[None, :, :, :]
    ey = py[:, None, None, :] - my[None, :, :, :]
    d2 = ex * ex + ey * ey
    kk = jax.lax.broadcasted_iota(jnp.int32, (8, 8, 8), 0)
    ii = jax.lax.broadcasted_iota(jnp.int32, (8, 8, 8), 1)
    jj = jax.lax.broadcasted_iota(jnp.int32, (8, 8, 8), 2)
    kmaskf = ((kk == ii) | (kk == jj)).astype(jnp.float32)
    s = d2 - r2[None, :, :, :] + kmaskf[:, :, :, None] * 1e30
    mins = jnp.min(s, axis=0)
    eye = (ii[0] == jj[0]).astype(jnp.float32)
    adj_f = (mins >= 0.0).astype(jnp.float32) * (1.0 - eye)[:, :, None]
    adj_ref[...] = adj_f

    a_hat = adj_f + eye[:, :, None]
    deg = jnp.sum(a_hat, axis=1)
    dinv = jax.lax.rsqrt(deg)
    norm = dinv[:, None, :] * a_hat * dinv[None, :, :]

    g1w = g1w_ref[...]
    g1b = g1b_ref[...]
    z1 = (px[:, None, :] * g1w[0][None, :, None]
          + py[:, None, :] * g1w[1][None, :, None])
    m1 = norm[:, 0, None, :] * z1[0][None, :, :]
    for j in range(1, 8):
        m1 = m1 + norm[:, j, None, :] * z1[j][None, :, :]
    h1 = jnp.maximum(m1 + g1b[0][None, :, None], 0.0)

    g2wt = g2wt_ref[...]
    g2b = g2b_ref[...]
    z2 = [jnp.dot(g2wt, h1[i], preferred_element_type=jnp.float32)
          for i in range(8)]
    h2 = norm[:, 0, None, :] * z2[0][None, :, :]
    for j in range(1, 8):
        h2 = h2 + norm[:, j, None, :] * z2[j][None, :, :]
    h2 = h2 + g2b[0][None, :, None]

    pooled = jnp.mean(h2, axis=0)
    dw1t = dw1t_ref[...]
    db1 = db1_ref[...]
    dh = jnp.maximum(jnp.dot(dw1t, pooled,
                             preferred_element_type=jnp.float32)
                     + db1[0][:, None], 0.0)
    dw2c = dw2c_ref[...]
    db2c = db2c_ref[...]
    rec_row = jnp.sum(dh * dw2c[0][:, None], axis=0) + db2c[0, 0]
    rec_ref[...] = jnp.broadcast_to(rec_row[None, :], (8, TB))

    lat_ref[0] = px
    lat_ref[1] = py


def kernel(x, enc_w1, enc_b1, enc_w2, enc_b2, gcn1_w, gcn1_b, gcn2_w, gcn2_b,
           dec_w1, dec_b1, dec_w2, dec_b2):
    B = x.shape[0]
    TB = _TB
    idx = jnp.arange(8, dtype=jnp.float32)

    xT = x.T
    encw1r = enc_w1[1:2]
    encc = idx[:, None] * enc_w1[2][None, :] + enc_b1[None, :]
    encw2t = enc_w2.T
    encb2 = enc_b2[:, None]
    g1b = gcn1_b[None, :]
    g2wt = gcn2_w.T
    g2b = gcn2_b[None, :]
    dw1t = dec_w1.T
    db1 = dec_b1[None, :]
    dw2c = dec_w2[:, 1][None, :]
    db2c = dec_b2[1].reshape(1, 1)

    grid = (B // TB,)

    def full(shape):
        nd = len(shape)
        return pl.BlockSpec(shape, lambda t, _n=nd: (0,) * _n)

    recT, lat, adjf = pl.pallas_call(
        _gae_kernel,
        grid=grid,
        in_specs=[
            pl.BlockSpec((8, TB), lambda t: (0, t)),
            full((1, 64)), full((8, 64)), full((2, 64)), full((2, 1)),
            full((2, 32)), full((1, 32)),
            full((32, 32)), full((1, 32)),
            full((64, 32)), full((1, 64)),
            full((1, 64)), full((1, 1)),
        ],
        out_specs=[
            pl.BlockSpec((8, TB), lambda t: (0, t)),
            pl.BlockSpec((2, 8, TB), lambda t: (0, 0, t)),
            pl.BlockSpec((8, 8, TB), lambda t: (0, 0, t)),
        ],
        out_shape=[
            jax.ShapeDtypeStruct((8, B), jnp.float32),
            jax.ShapeDtypeStruct((2, 8, B), jnp.float32),
            jax.ShapeDtypeStruct((8, 8, B), jnp.float32),
        ],
        compiler_params=pltpu.CompilerParams(
            dimension_semantics=("arbitrary",)),
    )(xT, encw1r, encc, encw2t, encb2,
      gcn1_w, g1b, g2wt, g2b, dw1t, db1, dw2c, db2c)

    rec = recT.T
    latent = lat.transpose(2, 1, 0)
    adj = adjf.transpose(2, 0, 1).astype(bool)
    return rec, latent, adj
